# baseline (device time: 63810 ns/iter reference)
import jax
import jax.numpy as jnp
from jax import lax
from jax.experimental import pallas as pl
from jax.experimental.pallas import tpu as pltpu

N_DEV = 4
BLK = 1024
HALF = BLK // 2
CHUNK_ORDER = ((1, 0), (3, 0), (1, 1), (3, 1), (2, 0), (2, 1))


def _sem_idx(d, c):
    return (d - 1) * 2 + c


def kernel(x, w_mat):
    k_full, m_per = x.shape
    k_full2, n = w_mat.shape
    assert k_full == k_full2

    def body(x_ref, w_ref, out_ref, comm_ref, xstage_ref, xloc_ref, sbuf_ref,
             wbuf_ref, acc_ref, send_sems, recv_sems, x_sems, loc_sem,
             w_sems, out_sems):
        my_i = lax.axis_index("i")
        tile_of_d = {1: 0, 3: 1, 2: 2}

        def start_stage(k):
            d, c = CHUNK_ORDER[k]
            dst = (my_i + d) % N_DEV
            cp = pltpu.make_async_copy(
                x_ref.at[pl.ds(dst * BLK + c * HALF, HALF), :],
                xstage_ref.at[k % 2],
                x_sems.at[k % 2],
            )
            cp.start()
            return cp

        stage_cps = {0: start_stage(0), 1: start_stage(1)}
        cp_loc = pltpu.make_async_copy(
            x_ref.at[pl.ds(my_i * BLK, BLK), :], xloc_ref, loc_sem
        )
        cp_loc.start()

        barrier_sem = pltpu.get_barrier_semaphore()
        for d in range(1, N_DEV):
            pl.semaphore_signal(
                barrier_sem, inc=1,
                device_id=((my_i + d) % N_DEV,),
                device_id_type=pl.DeviceIdType.MESH,
            )
        pl.semaphore_wait(barrier_sem, N_DEV - 1)

        rdmas = {}
        for k, (d, c) in enumerate(CHUNK_ORDER):
            t = tile_of_d[d]
            dst = (my_i + d) % N_DEV
            stage_cps[k].wait()
            sbuf_ref[t, pl.ds(c * HALF, HALF), :] = (
                xstage_ref[k % 2].astype(jnp.bfloat16)
            )
            if k + 2 < len(CHUNK_ORDER):
                stage_cps[k + 2] = start_stage(k + 2)
            rdma = pltpu.make_async_remote_copy(
                src_ref=sbuf_ref.at[t, pl.ds(c * HALF, HALF), :],
                dst_ref=comm_ref.at[d - 1, pl.ds(c * HALF, HALF), :],
                send_sem=send_sems.at[_sem_idx(d, c)],
                recv_sem=recv_sems.at[_sem_idx(d, c)],
                device_id=(dst,),
                device_id_type=pl.DeviceIdType.MESH,
            )
            rdma.start()
            rdmas[(d, c)] = rdma

        cp_w = []
        for slot, off in enumerate((0, 3, 1)):
            cp = pltpu.make_async_copy(
                w_ref.at[pl.ds(((my_i + off) % N_DEV) * BLK, BLK), :],
                wbuf_ref.at[slot],
                w_sems.at[slot],
            )
            cp.start()
            cp_w.append(cp)

        cp_loc.wait()
        cp_w[0].wait()
        acc_ref[:, :] = jnp.dot(
            xloc_ref[:, :], wbuf_ref[0], preferred_element_type=jnp.float32
        )
        cp_w3 = pltpu.make_async_copy(
            w_ref.at[pl.ds(((my_i + 2) % N_DEV) * BLK, BLK), :],
            wbuf_ref.at[0],
            w_sems.at[0],
        )
        cp_w3.start()

        def chunk_gemm(d, c, w_slot):
            rdmas[(d, c)].wait()
            rows = pl.ds(c * HALF, HALF)
            acc_ref[rows, :] += jnp.dot(
                comm_ref[d - 1, rows, :].astype(jnp.float32),
                wbuf_ref[w_slot],
                preferred_element_type=jnp.float32,
            )

        cp_w[1].wait()
        chunk_gemm(1, 0, 1)
        chunk_gemm(1, 1, 1)

        cp_w[2].wait()
        chunk_gemm(3, 0, 2)
        chunk_gemm(3, 1, 2)

        cp_w3.wait()
        cp_out = []
        for c in range(2):
            chunk_gemm(2, c, 0)
            rows = pl.ds(c * HALF, HALF)
            cp = pltpu.make_async_copy(
                acc_ref.at[rows, :], out_ref.at[rows, :], out_sems.at[c]
            )
            cp.start()
            cp_out.append(cp)
        for cp in cp_out:
            cp.wait()

    return pl.pallas_call(
        body,
        out_shape=jax.ShapeDtypeStruct((m_per, n), jnp.float32),
        in_specs=[
            pl.BlockSpec(memory_space=pltpu.MemorySpace.HBM),
            pl.BlockSpec(memory_space=pltpu.MemorySpace.HBM),
        ],
        out_specs=pl.BlockSpec(memory_space=pltpu.MemorySpace.HBM),
        scratch_shapes=[
            pltpu.VMEM((N_DEV - 1, BLK, m_per), jnp.bfloat16),
            pltpu.VMEM((2, HALF, m_per), jnp.float32),
            pltpu.VMEM((BLK, m_per), jnp.float32),
            pltpu.VMEM((N_DEV - 1, BLK, m_per), jnp.bfloat16),
            pltpu.VMEM((3, BLK, n), jnp.float32),
            pltpu.VMEM((BLK, n), jnp.float32),
            pltpu.SemaphoreType.DMA((6,)),
            pltpu.SemaphoreType.DMA((6,)),
            pltpu.SemaphoreType.DMA((2,)),
            pltpu.SemaphoreType.DMA,
            pltpu.SemaphoreType.DMA((3,)),
            pltpu.SemaphoreType.DMA((2,)),
        ],
        compiler_params=pltpu.CompilerParams(
            collective_id=0,
            vmem_limit_bytes=100 * 1024 * 1024,
        ),
    )(x, w_mat)


# device time: 60699 ns/iter; 1.0513x vs baseline; 1.0513x over previous
import jax
import jax.numpy as jnp
from jax import lax
from jax.experimental import pallas as pl
from jax.experimental.pallas import tpu as pltpu

N_DEV = 4
BLK = 1024
NCHUNK = 4
CH = BLK // NCHUNK
TILE_OF_D = {1: 0, 3: 1, 2: 2}
CHUNK_ORDER = tuple(
    (d, c) for c in range(NCHUNK) for d in (1, 3)
) + tuple((2, c) for c in range(NCHUNK))
N_STAGE = 3


def _sem_idx(d, c):
    return TILE_OF_D[d] * NCHUNK + c


def kernel(x, w_mat):
    k_full, m_per = x.shape
    k_full2, n = w_mat.shape
    assert k_full == k_full2

    def body(x_ref, w_ref, out_ref, comm_ref, xstage_ref, xloc_ref, sbuf_ref,
             wbuf_ref, acc_ref, send_sems, recv_sems, x_sems, loc_sem,
             w_sems, out_sems):
        my_i = lax.axis_index("i")

        def start_stage(k):
            d, c = CHUNK_ORDER[k]
            dst = (my_i + d) % N_DEV
            cp = pltpu.make_async_copy(
                x_ref.at[pl.ds(dst * BLK + c * CH, CH), :],
                xstage_ref.at[k % N_STAGE],
                x_sems.at[k % N_STAGE],
            )
            cp.start()
            return cp

        stage_cps = {k: start_stage(k) for k in range(N_STAGE)}
        cp_loc = pltpu.make_async_copy(
            x_ref.at[pl.ds(my_i * BLK, BLK), :], xloc_ref, loc_sem
        )
        cp_loc.start()

        barrier_sem = pltpu.get_barrier_semaphore()
        for d in range(1, N_DEV):
            pl.semaphore_signal(
                barrier_sem, inc=1,
                device_id=((my_i + d) % N_DEV,),
                device_id_type=pl.DeviceIdType.MESH,
            )
        pl.semaphore_wait(barrier_sem, N_DEV - 1)

        rdmas = {}
        for k, (d, c) in enumerate(CHUNK_ORDER):
            t = TILE_OF_D[d]
            dst = (my_i + d) % N_DEV
            stage_cps[k].wait()
            sbuf_ref[t, pl.ds(c * CH, CH), :] = (
                xstage_ref[k % N_STAGE].astype(jnp.bfloat16)
            )
            if k + N_STAGE < len(CHUNK_ORDER):
                stage_cps[k + N_STAGE] = start_stage(k + N_STAGE)
            rdma = pltpu.make_async_remote_copy(
                src_ref=sbuf_ref.at[t, pl.ds(c * CH, CH), :],
                dst_ref=comm_ref.at[d - 1, pl.ds(c * CH, CH), :],
                send_sem=send_sems.at[_sem_idx(d, c)],
                recv_sem=recv_sems.at[_sem_idx(d, c)],
                device_id=(dst,),
                device_id_type=pl.DeviceIdType.MESH,
            )
            rdma.start()
            rdmas[(d, c)] = rdma

        cp_w = []
        for slot, off in enumerate((0, 3, 1)):
            cp = pltpu.make_async_copy(
                w_ref.at[pl.ds(((my_i + off) % N_DEV) * BLK, BLK), :],
                wbuf_ref.at[slot],
                w_sems.at[slot],
            )
            cp.start()
            cp_w.append(cp)

        cp_loc.wait()
        cp_w[0].wait()
        acc_ref[:, :] = jnp.dot(
            xloc_ref[:, :], wbuf_ref[0], preferred_element_type=jnp.float32
        )
        cp_w3 = pltpu.make_async_copy(
            w_ref.at[pl.ds(((my_i + 2) % N_DEV) * BLK, BLK), :],
            wbuf_ref.at[0],
            w_sems.at[0],
        )
        cp_w3.start()

        def chunk_gemm(d, c, w_slot):
            rdmas[(d, c)].wait()
            rows = pl.ds(c * CH, CH)
            acc_ref[rows, :] += jnp.dot(
                comm_ref[d - 1, rows, :].astype(jnp.float32),
                wbuf_ref[w_slot],
                preferred_element_type=jnp.float32,
            )

        cp_w[1].wait()
        cp_w[2].wait()
        for c in range(NCHUNK):
            chunk_gemm(1, c, 1)
            chunk_gemm(3, c, 2)

        cp_w3.wait()
        cp_out = []
        for c in range(NCHUNK):
            chunk_gemm(2, c, 0)
            rows = pl.ds(c * CH, CH)
            cp = pltpu.make_async_copy(
                acc_ref.at[rows, :], out_ref.at[rows, :], out_sems.at[c]
            )
            cp.start()
            cp_out.append(cp)
        for cp in cp_out:
            cp.wait()

    return pl.pallas_call(
        body,
        out_shape=jax.ShapeDtypeStruct((m_per, n), jnp.float32),
        in_specs=[
            pl.BlockSpec(memory_space=pltpu.MemorySpace.HBM),
            pl.BlockSpec(memory_space=pltpu.MemorySpace.HBM),
        ],
        out_specs=pl.BlockSpec(memory_space=pltpu.MemorySpace.HBM),
        scratch_shapes=[
            pltpu.VMEM((N_DEV - 1, BLK, m_per), jnp.bfloat16),
            pltpu.VMEM((N_STAGE, CH, m_per), jnp.float32),
            pltpu.VMEM((BLK, m_per), jnp.float32),
            pltpu.VMEM((N_DEV - 1, BLK, m_per), jnp.bfloat16),
            pltpu.VMEM((3, BLK, n), jnp.float32),
            pltpu.VMEM((BLK, n), jnp.float32),
            pltpu.SemaphoreType.DMA((12,)),
            pltpu.SemaphoreType.DMA((12,)),
            pltpu.SemaphoreType.DMA((N_STAGE,)),
            pltpu.SemaphoreType.DMA,
            pltpu.SemaphoreType.DMA((3,)),
            pltpu.SemaphoreType.DMA((NCHUNK,)),
        ],
        compiler_params=pltpu.CompilerParams(
            collective_id=0,
            vmem_limit_bytes=100 * 1024 * 1024,
        ),
    )(x, w_mat)


# device time: 59643 ns/iter; 1.0699x vs baseline; 1.0177x over previous
import jax
import jax.numpy as jnp
from jax import lax
from jax.experimental import pallas as pl
from jax.experimental.pallas import tpu as pltpu

N_DEV = 4
BLK = 1024
NCHUNK = 8
CH = BLK // NCHUNK
TILE_OF_D = {1: 0, 3: 1, 2: 2}
CHUNK_ORDER = tuple(
    (d, c) for c in range(NCHUNK) for d in (1, 3)
) + tuple((2, c) for c in range(NCHUNK))
N_STAGE = 4


def _sem_idx(d, c):
    return TILE_OF_D[d] * NCHUNK + c


def kernel(x, w_mat):
    k_full, m_per = x.shape
    k_full2, n = w_mat.shape
    assert k_full == k_full2

    def body(x_ref, w_ref, out_ref, comm_ref, xstage_ref, xloc_ref, sbuf_ref,
             wbuf_ref, acc_ref, send_sems, recv_sems, x_sems, loc_sem,
             w_sems, out_sems):
        my_i = lax.axis_index("i")

        def start_stage(k):
            d, c = CHUNK_ORDER[k]
            dst = (my_i + d) % N_DEV
            cp = pltpu.make_async_copy(
                x_ref.at[pl.ds(dst * BLK + c * CH, CH), :],
                xstage_ref.at[k % N_STAGE],
                x_sems.at[k % N_STAGE],
            )
            cp.start()
            return cp

        stage_cps = {k: start_stage(k) for k in range(N_STAGE)}

        barrier_sem = pltpu.get_barrier_semaphore()
        for d in range(1, N_DEV):
            pl.semaphore_signal(
                barrier_sem, inc=1,
                device_id=((my_i + d) % N_DEV,),
                device_id_type=pl.DeviceIdType.MESH,
            )
        pl.semaphore_wait(barrier_sem, N_DEV - 1)

        rdmas = {}
        for k, (d, c) in enumerate(CHUNK_ORDER):
            t = TILE_OF_D[d]
            dst = (my_i + d) % N_DEV
            stage_cps[k].wait()
            sbuf_ref[t, pl.ds(c * CH, CH), :] = (
                xstage_ref[k % N_STAGE].astype(jnp.bfloat16)
            )
            if k + N_STAGE < len(CHUNK_ORDER):
                stage_cps[k + N_STAGE] = start_stage(k + N_STAGE)
            rdma = pltpu.make_async_remote_copy(
                src_ref=sbuf_ref.at[t, pl.ds(c * CH, CH), :],
                dst_ref=comm_ref.at[d - 1, pl.ds(c * CH, CH), :],
                send_sem=send_sems.at[_sem_idx(d, c)],
                recv_sem=recv_sems.at[_sem_idx(d, c)],
                device_id=(dst,),
                device_id_type=pl.DeviceIdType.MESH,
            )
            rdma.start()
            rdmas[(d, c)] = rdma

        cp_loc = pltpu.make_async_copy(
            x_ref.at[pl.ds(my_i * BLK, BLK), :], xloc_ref, loc_sem
        )
        cp_loc.start()

        cp_w = []
        for slot, off in enumerate((0, 3, 1)):
            cp = pltpu.make_async_copy(
                w_ref.at[pl.ds(((my_i + off) % N_DEV) * BLK, BLK), :],
                wbuf_ref.at[slot],
                w_sems.at[slot],
            )
            cp.start()
            cp_w.append(cp)

        cp_loc.wait()
        cp_w[0].wait()
        acc_ref[:, :] = jnp.dot(
            xloc_ref[:, :], wbuf_ref[0], preferred_element_type=jnp.float32
        )
        cp_w3 = pltpu.make_async_copy(
            w_ref.at[pl.ds(((my_i + 2) % N_DEV) * BLK, BLK), :],
            wbuf_ref.at[0],
            w_sems.at[0],
        )
        cp_w3.start()

        def chunk_gemm(d, c, w_slot):
            rdmas[(d, c)].wait()
            rows = pl.ds(c * CH, CH)
            acc_ref[rows, :] += jnp.dot(
                comm_ref[d - 1, rows, :].astype(jnp.float32),
                wbuf_ref[w_slot],
                preferred_element_type=jnp.float32,
            )

        cp_w[1].wait()
        cp_w[2].wait()
        for c in range(NCHUNK):
            chunk_gemm(1, c, 1)
            chunk_gemm(3, c, 2)

        cp_w3.wait()
        cp_out = []
        for c in range(NCHUNK):
            chunk_gemm(2, c, 0)
            rows = pl.ds(c * CH, CH)
            cp = pltpu.make_async_copy(
                acc_ref.at[rows, :], out_ref.at[rows, :], out_sems.at[c]
            )
            cp.start()
            cp_out.append(cp)
        for cp in cp_out:
            cp.wait()

    return pl.pallas_call(
        body,
        out_shape=jax.ShapeDtypeStruct((m_per, n), jnp.float32),
        in_specs=[
            pl.BlockSpec(memory_space=pltpu.MemorySpace.HBM),
            pl.BlockSpec(memory_space=pltpu.MemorySpace.HBM),
        ],
        out_specs=pl.BlockSpec(memory_space=pltpu.MemorySpace.HBM),
        scratch_shapes=[
            pltpu.VMEM((N_DEV - 1, BLK, m_per), jnp.bfloat16),
            pltpu.VMEM((N_STAGE, CH, m_per), jnp.float32),
            pltpu.VMEM((BLK, m_per), jnp.float32),
            pltpu.VMEM((N_DEV - 1, BLK, m_per), jnp.bfloat16),
            pltpu.VMEM((3, BLK, n), jnp.float32),
            pltpu.VMEM((BLK, n), jnp.float32),
            pltpu.SemaphoreType.DMA((24,)),
            pltpu.SemaphoreType.DMA((24,)),
            pltpu.SemaphoreType.DMA((N_STAGE,)),
            pltpu.SemaphoreType.DMA,
            pltpu.SemaphoreType.DMA((3,)),
            pltpu.SemaphoreType.DMA((NCHUNK,)),
        ],
        compiler_params=pltpu.CompilerParams(
            collective_id=0,
            vmem_limit_bytes=100 * 1024 * 1024,
        ),
    )(x, w_mat)
